# baseline (speedup 1.0000x reference)
import jax
import jax.numpy as jnp
from jax.experimental import pallas as pl

_LEVELS = 4
_K = 1024


def _rvq_body(f_ref, cb_ref, qsum_ref, idx_ref):
    r = f_ref[...]
    qsum = jnp.zeros_like(r)
    cols = jax.lax.broadcasted_iota(jnp.int32, (r.shape[0], _K), 1)
    idx_cols = []
    for lvl in range(_LEVELS):
        cb = cb_ref[lvl]
        b2 = jnp.sum(cb * cb, axis=1)
        a2 = jnp.sum(r * r, axis=1, keepdims=True)
        ab = jax.lax.dot_general(
            r, cb, (((1,), (1,)), ((), ())),
            precision=jax.lax.Precision.DEFAULT,
            preferred_element_type=jnp.float32)
        d = jnp.sqrt(jnp.maximum(a2 + b2[None, :] - 2.0 * ab, 0.0))
        m = jnp.min(d, axis=1, keepdims=True)
        idx = jnp.min(jnp.where(d == m, cols, jnp.int32(_K)),
                      axis=1, keepdims=True)
        onehot = (cols == idx).astype(jnp.float32)
        q = jax.lax.dot_general(
            onehot, cb, (((1,), (0,)), ((), ())),
            precision=jax.lax.Precision.HIGHEST,
            preferred_element_type=jnp.float32)
        qsum = qsum + q
        r = r - q
        idx_cols.append(idx)
    qsum_ref[...] = qsum
    idx_ref[...] = jnp.concatenate(idx_cols, axis=1)


def kernel(features, codebooks):
    n, d = features.shape
    levels, k, _ = codebooks.shape
    block = 512
    qsum, idx = pl.pallas_call(
        _rvq_body,
        grid=(n // block,),
        in_specs=[
            pl.BlockSpec((block, d), lambda i: (i, 0)),
            pl.BlockSpec((levels, k, d), lambda i: (0, 0, 0)),
        ],
        out_specs=[
            pl.BlockSpec((block, d), lambda i: (i, 0)),
            pl.BlockSpec((block, levels), lambda i: (i, 0)),
        ],
        out_shape=[
            jax.ShapeDtypeStruct((n, d), jnp.float32),
            jax.ShapeDtypeStruct((n, levels), jnp.int32),
        ],
    )(features, codebooks)
    return qsum, idx.T


# speedup vs baseline: 1.4835x; 1.4835x over previous
import jax
import jax.numpy as jnp
from jax.experimental import pallas as pl

_LEVELS = 4
_K = 1024


def _rvq_body(f_ref, cb_ref, cbh_ref, cbm_ref, cbl_ref, qsum_ref, idx_ref):
    r = f_ref[...]
    qsum = jnp.zeros_like(r)
    cols = jax.lax.broadcasted_iota(jnp.int32, (r.shape[0], _K), 1)
    idx_cols = []
    for lvl in range(_LEVELS):
        cb = cb_ref[lvl]
        b2 = jnp.sum(cb * cb, axis=1)
        a2 = jnp.sum(r * r, axis=1, keepdims=True)
        ab = jax.lax.dot_general(
            r, cb, (((1,), (1,)), ((), ())),
            precision=jax.lax.Precision.DEFAULT,
            preferred_element_type=jnp.float32)
        d = jnp.sqrt(jnp.maximum(a2 + b2[None, :] - 2.0 * ab, 0.0))
        m = jnp.min(d, axis=1, keepdims=True)
        idx = jnp.min(jnp.where(d == m, cols, jnp.int32(_K)),
                      axis=1, keepdims=True)
        onehot = (cols == idx).astype(jnp.bfloat16)
        q = jnp.zeros_like(r)
        for part_ref in (cbh_ref, cbm_ref, cbl_ref):
            q = q + jax.lax.dot_general(
                onehot, part_ref[lvl], (((1,), (0,)), ((), ())),
                precision=jax.lax.Precision.DEFAULT,
                preferred_element_type=jnp.float32)
        qsum = qsum + q
        r = r - q
        idx_cols.append(idx)
    qsum_ref[...] = qsum
    idx_ref[...] = jnp.concatenate(idx_cols, axis=1)


def kernel(features, codebooks):
    n, d = features.shape
    levels, k, _ = codebooks.shape
    mask = jnp.uint32(0xFFFF0000)

    def _trunc_bf16(x):
        u = jax.lax.bitcast_convert_type(x, jnp.uint32)
        return jax.lax.bitcast_convert_type(u & mask, jnp.float32)

    hi_f = _trunc_bf16(codebooks)
    mid_full = codebooks - hi_f
    mid_f = _trunc_bf16(mid_full)
    lo_f = mid_full - mid_f
    cbh = hi_f.astype(jnp.bfloat16)
    cbm = mid_f.astype(jnp.bfloat16)
    cbl = lo_f.astype(jnp.bfloat16)
    block = 512
    full = pl.BlockSpec((levels, k, d), lambda i: (0, 0, 0))
    qsum, idx = pl.pallas_call(
        _rvq_body,
        grid=(n // block,),
        in_specs=[pl.BlockSpec((block, d), lambda i: (i, 0)),
                  full, full, full, full],
        out_specs=[
            pl.BlockSpec((block, d), lambda i: (i, 0)),
            pl.BlockSpec((block, levels), lambda i: (i, 0)),
        ],
        out_shape=[
            jax.ShapeDtypeStruct((n, d), jnp.float32),
            jax.ShapeDtypeStruct((n, levels), jnp.int32),
        ],
    )(features, codebooks, cbh, cbm, cbl)
    return qsum, idx.T


# speedup vs baseline: 1.5269x; 1.0293x over previous
import jax
import jax.numpy as jnp
from jax.experimental import pallas as pl
from jax.experimental.pallas import tpu as pltpu

_LEVELS = 4
_K = 1024


def _rvq_body(f_ref, cb2_ref, cbp_ref, qsum_ref, idx_ref, b2_ref):
    @pl.when(pl.program_id(0) == 0)
    def _init_b2():
        for lvl in range(_LEVELS):
            c = cb2_ref[lvl] * 0.5
            b2_ref[pl.ds(lvl, 1), :] = jnp.sum(c * c, axis=1).reshape(1, _K)

    r = f_ref[...]
    b, d_dim = r.shape
    qsum = jnp.zeros_like(r)
    cols = jax.lax.broadcasted_iota(jnp.int32, (b, _K), 1)
    idx_cols = []
    for lvl in range(_LEVELS):
        a2 = jnp.sum(r * r, axis=1, keepdims=True)
        ab2 = jax.lax.dot_general(
            r, cb2_ref[lvl], (((1,), (1,)), ((), ())),
            precision=jax.lax.Precision.DEFAULT,
            preferred_element_type=jnp.float32)
        s = a2 + b2_ref[pl.ds(lvl, 1), :]
        d = jnp.sqrt(jnp.maximum(s - ab2, 0.0))
        idx = jnp.argmin(d, axis=1)[:, None]
        onehot = (cols == idx).astype(jnp.bfloat16)
        parts = jax.lax.dot_general(
            onehot, cbp_ref[lvl], (((1,), (0,)), ((), ())),
            precision=jax.lax.Precision.DEFAULT,
            preferred_element_type=jnp.float32)
        q = ((parts[:, :d_dim] + parts[:, d_dim:2 * d_dim])
             + parts[:, 2 * d_dim:])
        qsum = qsum + q
        r = r - q
        idx_cols.append(idx)
    qsum_ref[...] = qsum
    idx_ref[...] = jnp.concatenate(idx_cols, axis=1)


def kernel(features, codebooks):
    n, d = features.shape
    levels, k, _ = codebooks.shape
    mask = jnp.uint32(0xFFFF0000)

    def _trunc_bf16(x):
        u = jax.lax.bitcast_convert_type(x, jnp.uint32)
        return jax.lax.bitcast_convert_type(u & mask, jnp.float32)

    hi_f = _trunc_bf16(codebooks)
    mid_full = codebooks - hi_f
    mid_f = _trunc_bf16(mid_full)
    lo_f = mid_full - mid_f
    planes = jnp.concatenate([hi_f.astype(jnp.bfloat16),
                              mid_f.astype(jnp.bfloat16),
                              lo_f.astype(jnp.bfloat16)], axis=-1)
    block = 512
    qsum, idx = pl.pallas_call(
        _rvq_body,
        grid=(n // block,),
        in_specs=[
            pl.BlockSpec((block, d), lambda i: (i, 0)),
            pl.BlockSpec((levels, k, d), lambda i: (0, 0, 0)),
            pl.BlockSpec((levels, k, 3 * d), lambda i: (0, 0, 0)),
        ],
        out_specs=[
            pl.BlockSpec((block, d), lambda i: (i, 0)),
            pl.BlockSpec((block, levels), lambda i: (i, 0)),
        ],
        out_shape=[
            jax.ShapeDtypeStruct((n, d), jnp.float32),
            jax.ShapeDtypeStruct((n, levels), jnp.int32),
        ],
        scratch_shapes=[pltpu.VMEM((levels, k), jnp.float32)],
    )(features, codebooks * 2.0, planes)
    return qsum, idx.T


# speedup vs baseline: 1.6040x; 1.0505x over previous
import jax
import jax.numpy as jnp
from jax.experimental import pallas as pl
from jax.experimental.pallas import tpu as pltpu

_LEVELS = 4
_K = 1024


def _rvq_body(f_ref, cb2_ref, cbp_ref, qsum_ref, idx_ref, b2_ref):
    @pl.when(pl.program_id(0) == 0)
    def _init_b2():
        for lvl in range(_LEVELS):
            c = cb2_ref[lvl] * 0.5
            b2_ref[pl.ds(lvl, 1), :] = jnp.sum(c * c, axis=1).reshape(1, _K)

    r = f_ref[...]
    b, d_dim = r.shape
    qsum = jnp.zeros_like(r)
    cols = jax.lax.broadcasted_iota(jnp.int32, (b, _K), 1)
    idx_cols = []
    for lvl in range(_LEVELS):
        a2 = jnp.sum(r * r, axis=1, keepdims=True)
        ab2 = jax.lax.dot_general(
            r, cb2_ref[lvl], (((1,), (1,)), ((), ())),
            precision=jax.lax.Precision.DEFAULT,
            preferred_element_type=jnp.float32)
        s = a2 + b2_ref[pl.ds(lvl, 1), :]
        d = jnp.sqrt(jnp.maximum(s - ab2, 0.0))
        idx = jnp.argmin(d, axis=1)[:, None]
        onehot = (cols == idx).astype(jnp.bfloat16)
        parts = jax.lax.dot_general(
            onehot, cbp_ref[lvl], (((1,), (0,)), ((), ())),
            precision=jax.lax.Precision.DEFAULT,
            preferred_element_type=jnp.float32)
        q = ((parts[:, :d_dim] + parts[:, d_dim:2 * d_dim])
             + parts[:, 2 * d_dim:])
        qsum = qsum + q
        r = r - q
        idx_cols.append(idx)
    qsum_ref[...] = qsum
    idx_ref[...] = jnp.concatenate(idx_cols, axis=1)


def kernel(features, codebooks):
    n, d = features.shape
    levels, k, _ = codebooks.shape
    mask = jnp.uint32(0xFFFF0000)

    def _trunc_bf16(x):
        u = jax.lax.bitcast_convert_type(x, jnp.uint32)
        return jax.lax.bitcast_convert_type(u & mask, jnp.float32)

    hi_f = _trunc_bf16(codebooks)
    mid_full = codebooks - hi_f
    mid_f = _trunc_bf16(mid_full)
    lo_f = mid_full - mid_f
    planes = jnp.concatenate([hi_f.astype(jnp.bfloat16),
                              mid_f.astype(jnp.bfloat16),
                              lo_f.astype(jnp.bfloat16)], axis=-1)
    block = 1024
    qsum, idx = pl.pallas_call(
        _rvq_body,
        grid=(n // block,),
        in_specs=[
            pl.BlockSpec((block, d), lambda i: (i, 0)),
            pl.BlockSpec((levels, k, d), lambda i: (0, 0, 0)),
            pl.BlockSpec((levels, k, 3 * d), lambda i: (0, 0, 0)),
        ],
        out_specs=[
            pl.BlockSpec((block, d), lambda i: (i, 0)),
            pl.BlockSpec((block, levels), lambda i: (i, 0)),
        ],
        out_shape=[
            jax.ShapeDtypeStruct((n, d), jnp.float32),
            jax.ShapeDtypeStruct((n, levels), jnp.int32),
        ],
        scratch_shapes=[pltpu.VMEM((levels, k), jnp.float32)],
    )(features, codebooks * 2.0, planes)
    return qsum, idx.T
